# baseline (speedup 1.0000x reference)
import functools

import jax
import jax.numpy as jnp
from jax import lax
from jax.experimental import pallas as pl
from jax.experimental.pallas import tpu as pltpu
from jax.experimental.pallas import tpu_sc as plsc

B = 16384
D = 64
C = 100000
SCALE = 0.05

NC = 2
NS = 16
PASSES = D // (NC * NS)


def _body(ctr_t, feat_t, lab_hbm, flg_hbm, out_t, loss_hbm,
          acc_v, f_v, lab_v, part_v, flg_v, flg_s, lab_sh, sem, semw):
    cid = lax.axis_index("c")
    sid = lax.axis_index("s")

    pltpu.sync_copy(flg_hbm, flg_v)
    for k in range(16):
        f16 = flg_v[pl.ds(k * 16, 16)]
        for j in range(16):
            flg_s[k * 16 + j] = f16[j]
    for j in range(8):
        r = sid * 8 + j
        pltpu.sync_copy(lab_hbm.at[pl.ds(r * 128, 128)],
                        lab_sh.at[pl.ds(r * 128, 128)])
    plsc.subcore_barrier()

    iota = lax.iota(jnp.int32, 16)
    sqs = [jnp.zeros((16,), jnp.float32) for _ in range(4)]
    cp_w = None
    for p in range(PASSES):
        d = cid * (PASSES * NS) + p * NS + sid
        if cp_w is not None:
            cp_w.wait()
        cp_a = pltpu.async_copy(ctr_t.at[d], acc_v, sem)
        cp_f = pltpu.async_copy(feat_t.at[d], f_v, sem)
        cp_a.wait()
        cp_f.wait()

        cps = pltpu.async_copy(lab_sh.at[pl.ds(0, 2048)], lab_v.at[0], sem)
        for ch in range(8):
            cps.wait()
            if ch < 7:
                cps = pltpu.async_copy(lab_sh.at[pl.ds((ch + 1) * 2048, 2048)],
                                       lab_v.at[(ch + 1) & 1], sem)
            buf = ch & 1

            def grp_a(g, sqs, ch=ch, buf=buf):
                sqs = list(sqs)
                labv = lab_v[buf, pl.ds(g * 16, 16)]
                s0 = ch * 2048 + g * 16
                f16 = f_v[pl.ds(s0, 16)]
                c16 = plsc.load_gather(acc_v, [labv])
                d16 = f16 - c16
                sqs[0] = sqs[0] + d16 * d16
                f_v[pl.ds(s0, 16)] = d16 * SCALE
                return tuple(sqs)

            sqs = list(plsc.parallel_loop(0, 128, 1, unroll=8,
                                          carry=tuple(sqs))(grp_a))

        cps = pltpu.async_copy(lab_sh.at[pl.ds(0, 2048)], lab_v.at[0], sem)
        for ch in range(8):
            cps.wait()
            if ch < 7:
                cps = pltpu.async_copy(lab_sh.at[pl.ds((ch + 1) * 2048, 2048)],
                                       lab_v.at[(ch + 1) & 1], sem)
            buf = ch & 1

            def grp_b(qq, carry, ch=ch, buf=buf):
                for h in range(2):
                    q = qq * 2 + h
                    labvs, deltas = [], []
                    for u in range(4):
                        g = q * 4 + u
                        labvs.append(lab_v[buf, pl.ds(g * 16, 16)])
                        deltas.append(f_v[pl.ds(ch * 2048 + g * 16, 16)])
                    flag = flg_s[ch * 32 + q]

                    @pl.when(flag == 0)
                    def _(labvs=labvs, deltas=deltas):
                        for u in range(4):
                            plsc.addupdate_scatter(acc_v, [labvs[u]],
                                                   deltas[u])

                    @pl.when(flag != 0)
                    def _(labvs=labvs, deltas=deltas):
                        for u in range(4):
                            for j in range(16):
                                plsc.addupdate_scatter(acc_v, [labvs[u]],
                                                       deltas[u],
                                                       mask=iota == j)
                return carry

            lax.fori_loop(0, 16, grp_b, 0)

        cp_w = pltpu.async_copy(acc_v, out_t.at[d], semw)
    cp_w.wait()
    part_v[...] = (sqs[0] + sqs[1]) + (sqs[2] + sqs[3])
    wid = cid * NS + sid
    pltpu.sync_copy(part_v, loss_hbm.at[wid])


_sc_call = functools.partial(
    pl.kernel,
    out_type=(jax.ShapeDtypeStruct((D, C), jnp.float32),
              jax.ShapeDtypeStruct((NC * NS, 16), jnp.float32)),
    mesh=plsc.VectorSubcoreMesh(core_axis_name="c", subcore_axis_name="s",
                                num_cores=NC, num_subcores=NS),
    scratch_types=[
        pltpu.VMEM((C,), jnp.float32),
        pltpu.VMEM((B,), jnp.float32),
        pltpu.VMEM((2, 2048), jnp.int32),
        pltpu.VMEM((16,), jnp.float32),
        pltpu.VMEM((256,), jnp.int32),
        pltpu.SMEM((256,), jnp.int32),
        pltpu.VMEM_SHARED((B,), jnp.int32),
        pltpu.SemaphoreType.DMA,
        pltpu.SemaphoreType.DMA,
    ],
    compiler_params=pltpu.CompilerParams(needs_layout_passes=False),
)(_body)


def kernel(features, labels, center_var):
    labels = labels.reshape(-1)
    lab2 = labels.reshape(1024, 16)
    eq = lab2[:, :, None] == lab2[:, None, :]
    pair = jnp.triu(jnp.ones((16, 16), jnp.bool_), k=1)
    grp_dup = jnp.any(jnp.logical_and(eq, pair), axis=(1, 2))
    flags = jnp.any(grp_dup.reshape(256, 4), axis=1).astype(jnp.int32)
    out_t, parts = _sc_call(center_var.T, features.T, labels, flags)
    loss = jnp.sum(parts) * (1.0 / (B * D))
    return loss, out_t.T
